# baseline (speedup 1.0000x reference)
import functools

import jax
import jax.numpy as jnp
from jax import lax
from jax.experimental import pallas as pl
from jax.experimental.pallas import tpu as pltpu
from jax.experimental.pallas import tpu_sc as plsc

_B, _N, _PD, _C, _K, _P = 4, 4096, 3, 32, 16, 32
_H1 = 2 * _P
_R = 256
_NC, _NS = 2, 16
_NW = _NC * _NS
_GC = 128
_CH = (_B * _N * _K) // (_NW * _GC)
_RN3 = 128


def _select_body(pts_ref, ptsT_ref, feat_ref, w1a_ref, w1g_ref, b1_ref,
                 idx_ref, fg_ref):
    b = pl.program_id(0)
    p = pts_ref[0]
    pT = ptsT_ref[0]
    m = jnp.dot(p, pT, preferred_element_type=jnp.float32)
    r = jnp.sum(p * p, axis=1, keepdims=True)
    c = jnp.sum(pT * pT, axis=0, keepdims=True)
    D = r - 2.0 * m + c + 1e-5

    fiota = lax.broadcasted_iota(jnp.int32, (_R, _N), 1).astype(jnp.float32)
    kiota = lax.broadcasted_iota(jnp.int32, (_R, _K), 1)
    acc = jnp.zeros((_R, _K), dtype=jnp.float32)
    big = jnp.float32(jnp.inf)
    bigf = jnp.float32(1e30)
    for j in range(_K + 1):
        v = jnp.min(D, axis=1, keepdims=True)
        cand = jnp.where(D == v, fiota, bigf)
        amin = jnp.min(cand, axis=1, keepdims=True)
        D = jnp.where(cand == amin, big, D)
        if j > 0:
            acc = jnp.where(kiota == (j - 1), amin, acc)
    idx_ref[...] = acc.astype(jnp.int32) + b * _N

    f = feat_ref[0]
    f1 = jnp.dot(f, w1a_ref[...], preferred_element_type=jnp.float32)
    g1 = (jnp.dot(f, w1g_ref[...], preferred_element_type=jnp.float32)
          + b1_ref[...])
    fg_ref[...] = jnp.concatenate([f1, g1], axis=-1)


def _gelu(x):
    return x * (0.5 * (1.0 + lax.erf(x * 0.7071067811865476)))


def _mlp_body(h_ref, fg_ref, w2_ref, b2_ref, out_ref):
    h = h_ref[...][:, :_H1]
    g = fg_ref[...][:, _H1:]
    x = h.reshape(_RN3, _K, _H1) + g[:, None, :]
    x = _gelu(x)
    y = jnp.dot(x.reshape(_RN3 * _K, _H1), w2_ref[...],
                preferred_element_type=jnp.float32) + b2_ref[...]
    y = _gelu(y)
    out_ref[...] = jnp.sum(y.reshape(_RN3, _K, _P), axis=1) * (1.0 / _K)


def _sc_gather_body(f1_hbm, idx_hbm, out_hbm, idx_v, rows_v, sem):
    wid = lax.axis_index("s") * _NC + lax.axis_index("c")
    pltpu.sync_copy(idx_hbm.at[pl.ds(wid * _CH, _CH)], idx_v)

    def body(j, carry):
        pltpu.async_copy(f1_hbm.at[idx_v.at[j]], rows_v, sem).wait()
        pltpu.sync_copy(rows_v,
                        out_hbm.at[pl.ds(wid * _CH * _GC + j * _GC, _GC)])
        return carry

    lax.fori_loop(0, _CH, body, 0)


def kernel(points, features, W1, b1, W2, b2):
    f32 = jnp.float32
    pts_pad = jnp.concatenate(
        [points, jnp.zeros((_B, _N, 8 - _PD), f32)], axis=-1)
    ptsT = jnp.swapaxes(pts_pad, 1, 2)
    W1a = W1[:_C]
    W1g = W1[_C:] - W1a

    grid1 = (_B, _N // _R)
    idx, FG = pl.pallas_call(
        _select_body,
        grid=grid1,
        in_specs=[
            pl.BlockSpec((1, _R, 8), lambda b, i: (b, i, 0)),
            pl.BlockSpec((1, 8, _N), lambda b, i: (b, 0, 0)),
            pl.BlockSpec((1, _R, _C), lambda b, i: (b, i, 0)),
            pl.BlockSpec((_C, _H1), lambda b, i: (0, 0)),
            pl.BlockSpec((_C, _H1), lambda b, i: (0, 0)),
            pl.BlockSpec((1, _H1), lambda b, i: (0, 0)),
        ],
        out_specs=[
            pl.BlockSpec((_R, _K), lambda b, i: (b * (_N // _R) + i, 0)),
            pl.BlockSpec((_R, 2 * _H1), lambda b, i: (b * (_N // _R) + i, 0)),
        ],
        out_shape=[
            jax.ShapeDtypeStruct((_B * _N, _K), jnp.int32),
            jax.ShapeDtypeStruct((_B * _N, 2 * _H1), f32),
        ],
    )(pts_pad, ptsT, features, W1a, W1g, b1.reshape(1, _H1))

    mesh = plsc.VectorSubcoreMesh(core_axis_name="c", subcore_axis_name="s",
                                  num_cores=_NC, num_subcores=_NS)
    sc_gather = functools.partial(
        pl.kernel,
        out_type=jax.ShapeDtypeStruct((_B * _N * _K, 2 * _H1), f32),
        mesh=mesh,
        scratch_types=[
            pltpu.VMEM((_CH, _GC), jnp.int32),
            pltpu.VMEM((_GC, 2 * _H1), f32),
            pltpu.SemaphoreType.DMA,
        ],
    )(_sc_gather_body)
    H = sc_gather(FG, idx.reshape(_NW * _CH, _GC))

    grid3 = ((_B * _N) // _RN3,)
    out = pl.pallas_call(
        _mlp_body,
        grid=grid3,
        in_specs=[
            pl.BlockSpec((_RN3 * _K, 2 * _H1), lambda i: (i, 0)),
            pl.BlockSpec((_RN3, 2 * _H1), lambda i: (i, 0)),
            pl.BlockSpec((_H1, _P), lambda i: (0, 0)),
            pl.BlockSpec((1, _P), lambda i: (0, 0)),
        ],
        out_specs=pl.BlockSpec((_RN3, _P), lambda i: (i, 0)),
        out_shape=jax.ShapeDtypeStruct((_B * _N, _P), f32),
    )(H, FG, W2, b2.reshape(1, _P))

    return out.reshape(_B, _N, _P)


# speedup vs baseline: 1.0901x; 1.0901x over previous
import functools

import jax
import jax.numpy as jnp
from jax import lax
from jax.experimental import pallas as pl
from jax.experimental.pallas import tpu as pltpu
from jax.experimental.pallas import tpu_sc as plsc

_B, _N, _PD, _C, _K, _P = 4, 4096, 3, 32, 16, 32
_H1 = 2 * _P
_R = 256
_NC, _NS = 2, 16
_NW = _NC * _NS
_GC = 128
_CH = (_B * _N * _K) // (_NW * _GC)
_RN3 = 128


def _select_body(pts_ref, ptsT_ref, feat_ref, w1a_ref, w1g_ref, b1_ref,
                 idx_ref, fg_ref):
    b = pl.program_id(0)
    p = pts_ref[0]
    pT = ptsT_ref[0]
    m = jnp.dot(p, pT, preferred_element_type=jnp.float32)
    r = jnp.sum(p * p, axis=1, keepdims=True)
    c = jnp.sum(pT * pT, axis=0, keepdims=True)
    D = r - 2.0 * m + c + 1e-5

    fiota = lax.broadcasted_iota(jnp.int32, (_R, _N), 1).astype(jnp.float32)
    kiota = lax.broadcasted_iota(jnp.int32, (_R, _K), 1)
    acc = jnp.zeros((_R, _K), dtype=jnp.float32)
    big = jnp.float32(jnp.inf)
    bigf = jnp.float32(1e30)
    for j in range(_K + 1):
        v = jnp.min(D, axis=1, keepdims=True)
        cand = jnp.where(D == v, fiota, bigf)
        amin = jnp.min(cand, axis=1, keepdims=True)
        D = jnp.where(cand == amin, big, D)
        if j > 0:
            acc = jnp.where(kiota == (j - 1), amin, acc)
    idx_ref[...] = acc.astype(jnp.int32) + b * _N

    f = feat_ref[0]
    f1 = jnp.dot(f, w1a_ref[...], preferred_element_type=jnp.float32)
    g1 = (jnp.dot(f, w1g_ref[...], preferred_element_type=jnp.float32)
          + b1_ref[...])
    fg_ref[...] = jnp.concatenate([f1, g1], axis=-1)


def _gelu(x):
    return x * (0.5 * (1.0 + lax.erf(x * 0.7071067811865476)))


def _mlp_body(h_ref, fg_ref, w2_ref, b2_ref, out_ref):
    h = h_ref[...][:, :_H1]
    g = fg_ref[...][:, _H1:]
    x = h.reshape(_RN3, _K, _H1) + g[:, None, :]
    x = _gelu(x)
    y = jnp.dot(x.reshape(_RN3 * _K, _H1), w2_ref[...],
                preferred_element_type=jnp.float32) + b2_ref[...]
    y = _gelu(y)
    out_ref[...] = jnp.sum(y.reshape(_RN3, _K, _P), axis=1) * (1.0 / _K)


def _sc_gather_body(f1_hbm, idx_hbm, out_hbm, idx_v, rows_v, sem):
    ch = idx_v.shape[0]
    wid = lax.axis_index("s") * _NC + lax.axis_index("c")
    pltpu.sync_copy(idx_hbm.at[pl.ds(wid * ch, ch)], idx_v)

    def body(j, carry):
        pltpu.async_copy(f1_hbm.at[idx_v.at[j]], rows_v, sem).wait()
        pltpu.sync_copy(rows_v,
                        out_hbm.at[pl.ds(wid * ch * _GC + j * _GC, _GC)])
        return carry

    lax.fori_loop(0, ch, body, 0)


def _half_chain(points_h, features_h, W1a, W1g, b1r, W2, b2r, mesh):
    f32 = jnp.float32
    bh = points_h.shape[0]
    pts_pad = jnp.concatenate(
        [points_h, jnp.zeros((bh, _N, 8 - _PD), f32)], axis=-1)
    ptsT = jnp.swapaxes(pts_pad, 1, 2)

    grid1 = (bh, _N // _R)
    idx, FG = pl.pallas_call(
        _select_body,
        grid=grid1,
        in_specs=[
            pl.BlockSpec((1, _R, 8), lambda b, i: (b, i, 0)),
            pl.BlockSpec((1, 8, _N), lambda b, i: (b, 0, 0)),
            pl.BlockSpec((1, _R, _C), lambda b, i: (b, i, 0)),
            pl.BlockSpec((_C, _H1), lambda b, i: (0, 0)),
            pl.BlockSpec((_C, _H1), lambda b, i: (0, 0)),
            pl.BlockSpec((1, _H1), lambda b, i: (0, 0)),
        ],
        out_specs=[
            pl.BlockSpec((_R, _K), lambda b, i: (b * (_N // _R) + i, 0)),
            pl.BlockSpec((_R, 2 * _H1), lambda b, i: (b * (_N // _R) + i, 0)),
        ],
        out_shape=[
            jax.ShapeDtypeStruct((bh * _N, _K), jnp.int32),
            jax.ShapeDtypeStruct((bh * _N, 2 * _H1), f32),
        ],
    )(pts_pad, ptsT, features_h, W1a, W1g, b1r)

    ch = (bh * _N * _K) // (_NW * _GC)
    sc_gather = functools.partial(
        pl.kernel,
        out_type=jax.ShapeDtypeStruct((bh * _N * _K, 2 * _H1), f32),
        mesh=mesh,
        scratch_types=[
            pltpu.VMEM((ch, _GC), jnp.int32),
            pltpu.VMEM((_GC, 2 * _H1), f32),
            pltpu.SemaphoreType.DMA,
        ],
    )(_sc_gather_body)
    H = sc_gather(FG, idx.reshape(_NW * ch, _GC))

    grid3 = ((bh * _N) // _RN3,)
    out = pl.pallas_call(
        _mlp_body,
        grid=grid3,
        in_specs=[
            pl.BlockSpec((_RN3 * _K, 2 * _H1), lambda i: (i, 0)),
            pl.BlockSpec((_RN3, 2 * _H1), lambda i: (i, 0)),
            pl.BlockSpec((_H1, _P), lambda i: (0, 0)),
            pl.BlockSpec((1, _P), lambda i: (0, 0)),
        ],
        out_specs=pl.BlockSpec((_RN3, _P), lambda i: (i, 0)),
        out_shape=jax.ShapeDtypeStruct((bh * _N, _P), f32),
    )(H, FG, W2, b2r)
    return out


def kernel(points, features, W1, b1, W2, b2):
    W1a = W1[:_C]
    W1g = W1[_C:] - W1a
    b1r = b1.reshape(1, _H1)
    b2r = b2.reshape(1, _P)
    mesh = plsc.VectorSubcoreMesh(core_axis_name="c", subcore_axis_name="s",
                                  num_cores=_NC, num_subcores=_NS)
    hb = _B // 2
    out0 = _half_chain(points[:hb], features[:hb], W1a, W1g, b1r, W2, b2r, mesh)
    out1 = _half_chain(points[hb:], features[hb:], W1a, W1g, b1r, W2, b2r, mesh)
    return jnp.concatenate([out0, out1], axis=0).reshape(_B, _N, _P)
